# baseline (speedup 1.0000x reference)
import functools

import jax
import jax.numpy as jnp
from jax import lax
from jax.experimental import pallas as pl
from jax.experimental.pallas import tpu as pltpu
from jax.experimental.pallas import tpu_sc as plsc

_NC = 2
_NS = 16
_NW = _NC * _NS
_CHUNK = 128


def _make_sc_gather(batch, emb_dim, n_chunks):
    mesh = plsc.VectorSubcoreMesh(core_axis_name="c", subcore_axis_name="s")

    @functools.partial(
        pl.kernel,
        out_type=jax.ShapeDtypeStruct((batch, emb_dim), jnp.float32),
        mesh=mesh,
        scratch_types=[
            pltpu.VMEM((n_chunks, _CHUNK), jnp.int32),
            pltpu.VMEM((n_chunks, _CHUNK, emb_dim), jnp.float32),
            pltpu.SemaphoreType.DMA,
        ],
        compiler_params=pltpu.CompilerParams(use_tc_tiling_on_sc=False),
    )
    def gather(table_hbm, idx_hbm, out_hbm, idx_v, rows_v, sem):
        wid = lax.axis_index("s") * _NC + lax.axis_index("c")
        base = wid * (n_chunks * _CHUNK)
        pltpu.sync_copy(idx_hbm.at[wid], idx_v)
        copies = [
            pltpu.async_copy(table_hbm.at[idx_v.at[j]], rows_v.at[j], sem)
            for j in range(n_chunks)
        ]
        for j, c in enumerate(copies):
            c.wait()
            pltpu.sync_copy(rows_v.at[j], out_hbm.at[pl.ds(base + j * _CHUNK, _CHUNK)])

    return gather


def _mlp_body(x_ref, w1_ref, b1_ref, w2_ref, b2_ref, o_ref):
    x = x_ref[...]
    h = jnp.dot(x, w1_ref[...], preferred_element_type=jnp.float32) + b1_ref[...]
    h = jnp.maximum(h, 0.0)
    o_ref[...] = jnp.dot(h, w2_ref[...], preferred_element_type=jnp.float32) + b2_ref[...]


def kernel(user_id, table, W1, b1, W2, b2):
    batch = user_id.shape[0]
    emb_dim = table.shape[1]
    n_chunks = batch // (_NW * _CHUNK)

    idx = user_id.astype(jnp.int32).reshape(_NW, n_chunks, _CHUNK)
    emb = _make_sc_gather(batch, emb_dim, n_chunks)(table, idx)

    block = 2048
    grid = batch // block
    out = pl.pallas_call(
        _mlp_body,
        grid=(grid,),
        in_specs=[
            pl.BlockSpec((block, emb_dim), lambda i: (i, 0)),
            pl.BlockSpec((emb_dim, emb_dim), lambda i: (0, 0)),
            pl.BlockSpec((1, emb_dim), lambda i: (0, 0)),
            pl.BlockSpec((emb_dim, emb_dim), lambda i: (0, 0)),
            pl.BlockSpec((1, emb_dim), lambda i: (0, 0)),
        ],
        out_specs=pl.BlockSpec((block, emb_dim), lambda i: (i, 0)),
        out_shape=jax.ShapeDtypeStruct((batch, emb_dim), jnp.float32),
    )(emb, W1, b1.reshape(1, emb_dim), W2, b2.reshape(1, emb_dim))
    return out


# speedup vs baseline: 2.4724x; 2.4724x over previous
import functools

import jax
import jax.numpy as jnp
from jax import lax
from jax.experimental import pallas as pl
from jax.experimental.pallas import tpu as pltpu
from jax.experimental.pallas import tpu_sc as plsc

_NC = 2
_NS = 16
_NW = _NC * _NS
_L = 16

_ROWS = 1000001
_BATCH = 16384
_CHUNK = 512
_CH_PER_W = 61
_R_PER_W = _CH_PER_W * _CHUNK
_MAIN_END = _NW * _R_PER_W
_E1_LO = _MAIN_END
_E2_LO = _MAIN_END + _CHUNK
_E2_W = _ROWS - _E2_LO

_HITS_CAP = 1024
_HITS_DUMP = _HITS_CAP + _L
_WORK_CAP = 128
_WORK_DUMP = _WORK_CAP + _L
_STAGE = 128


def _lanes():
    return lax.broadcasted_iota(jnp.int32, (_L,), 0)


def _splat(x):
    return jnp.full((_L,), x, dtype=jnp.int32)


def _scalar_at(ref, w):
    base = (w // _L) * _L
    vec = ref[pl.ds(base, _L)]
    lane = w - base
    return jnp.max(jnp.where(_lanes() == lane, vec, jnp.int32(-1)))


def _collect_hits(idx_v, hits_r, hits_b, lo, hi, count):
    def body(g, cnt):
        v = idx_v[pl.ds(g * _L, _L)]
        m = (v >= lo) & (v < hi)
        mi = m.astype(jnp.int32)
        pos = jnp.where(m, cnt - 1 + jnp.cumsum(mi), _HITS_DUMP)
        plsc.store_scatter(hits_r, [pos], v)
        plsc.store_scatter(hits_b, [pos], _lanes() + g * _L)
        return jnp.minimum(cnt + jnp.sum(mi), _HITS_CAP)

    return lax.fori_loop(0, _BATCH // _L, body, count)


def _reset_scatteridx(scatteridx):
    for k in range(_STAGE // _L):
        scatteridx[pl.ds(k * _L, _L)] = _splat(_BATCH)


def _process_window(buf, win_lo, win_w, state, refs):
    (hits_r, hits_b, work_m, work_b, rowstage, scatteridx, out_hbm,
     scat_sem) = refs
    count, nrows = state

    def scan_body(h, wc):
        hv = hits_r[pl.ds(h * _L, _L)]
        bv = hits_b[pl.ds(h * _L, _L)]
        m = (hv >= win_lo) & (hv < win_lo + win_w)
        mi = m.astype(jnp.int32)
        pos = jnp.where(m, wc - 1 + jnp.cumsum(mi), _WORK_DUMP)
        plsc.store_scatter(work_m, [pos], hv - win_lo)
        plsc.store_scatter(work_b, [pos], bv)
        return jnp.minimum(wc + jnp.sum(mi), _WORK_CAP)

    n_win = (count + _L - 1) // _L
    wc = lax.fori_loop(0, n_win, scan_body, jnp.int32(0))

    def hit_body(w, nr):
        m_loc = _scalar_at(work_m, w)
        b = _scalar_at(work_b, w)
        for g in range(4):
            cvec = _lanes() + g * _L
            vals = plsc.load_gather(buf, [cvec, _splat(m_loc)])
            rowstage[nr, pl.ds(g * _L, _L)] = vals
        sbase = (nr // _L) * _L
        svec = scatteridx[pl.ds(sbase, _L)]
        scatteridx[pl.ds(sbase, _L)] = jnp.where(
            _lanes() == nr - sbase, b, svec
        )
        full = nr + 1 == _STAGE

        @pl.when(full)
        def _():
            pltpu.async_copy(rowstage, out_hbm.at[scatteridx], scat_sem).wait()
            _reset_scatteridx(scatteridx)

        return jnp.where(full, jnp.int32(0), nr + 1)

    nrows = lax.fori_loop(0, wc, hit_body, nrows)
    return count, nrows


def _make_sc_gather(batch, emb_dim):
    mesh = plsc.VectorSubcoreMesh(core_axis_name="c", subcore_axis_name="s")

    @functools.partial(
        pl.kernel,
        out_type=jax.ShapeDtypeStruct((batch + 8, 2 * emb_dim), jnp.float32),
        mesh=mesh,
        scratch_types=[
            pltpu.VMEM((batch,), jnp.int32),
            pltpu.VMEM((2, emb_dim, _CHUNK), jnp.float32),
            pltpu.VMEM((emb_dim, _E2_W), jnp.float32),
            pltpu.VMEM((_HITS_DUMP + _L,), jnp.int32),
            pltpu.VMEM((_HITS_DUMP + _L,), jnp.int32),
            pltpu.VMEM((_WORK_DUMP + _L,), jnp.int32),
            pltpu.VMEM((_WORK_DUMP + _L,), jnp.int32),
            pltpu.VMEM((_STAGE, 2 * emb_dim), jnp.float32),
            pltpu.VMEM((_STAGE,), jnp.int32),
            pltpu.SemaphoreType.DMA,
            pltpu.SemaphoreType.DMA,
        ],
        compiler_params=pltpu.CompilerParams(needs_layout_passes=False),
    )
    def gather(table_t_hbm, idx_hbm, out_hbm, idx_v, buf, tail_buf, hits_r,
               hits_b, work_m, work_b, rowstage, scatteridx, stream_sem,
               scat_sem):
        wid = lax.axis_index("s") * _NC + lax.axis_index("c")
        base = wid * _R_PER_W

        pltpu.sync_copy(idx_hbm, idx_v)
        _reset_scatteridx(scatteridx)
        for k in range(0, _HITS_DUMP + _L, _L):
            hits_r[pl.ds(k, _L)] = _splat(jnp.int32(2**31 - 1))

        count = _collect_hits(idx_v, hits_r, hits_b, base,
                              base + _R_PER_W, jnp.int32(0))
        lo_x = jnp.where(wid == 0, _E1_LO, jnp.where(wid == 1, _E2_LO, 0))
        hi_x = jnp.where(wid == 0, _E2_LO, jnp.where(wid == 1, _ROWS, 0))
        count = _collect_hits(idx_v, hits_r, hits_b, lo_x, hi_x, count)

        n_chunks = jnp.where(wid == 0, _CH_PER_W + 1, _CH_PER_W)

        def chunk_lo(c):
            return jnp.where(c < _CH_PER_W, base + c * _CHUNK, _E1_LO)

        refs = (hits_r, hits_b, work_m, work_b, rowstage, scatteridx,
                out_hbm, scat_sem)

        pltpu.async_copy(
            table_t_hbm.at[:, pl.ds(chunk_lo(jnp.int32(0)), _CHUNK)],
            buf.at[0], stream_sem,
        )

        def chunk_body(c, state):
            pltpu.make_async_copy(
                table_t_hbm.at[:, pl.ds(0, _CHUNK)], buf.at[c % 2], stream_sem
            ).wait()

            @pl.when(c + 1 < n_chunks)
            def _():
                pltpu.async_copy(
                    table_t_hbm.at[:, pl.ds(chunk_lo(c + 1), _CHUNK)],
                    buf.at[(c + 1) % 2], stream_sem,
                )

            return _process_window(buf.at[c % 2], chunk_lo(c), _CHUNK, state,
                                   refs)

        count, nrows = lax.fori_loop(0, n_chunks, chunk_body,
                                     (count, jnp.int32(0)))

        pltpu.sync_copy(table_t_hbm.at[:, pl.ds(_E2_LO, _E2_W)], tail_buf)
        _, nrows = _process_window(tail_buf, _E2_LO, _E2_W, (count, nrows),
                                   refs)

        @pl.when(nrows > 0)
        def _():
            pltpu.async_copy(rowstage, out_hbm.at[scatteridx], scat_sem).wait()

    return gather


def _mlp_body(x_ref, w1_ref, b1_ref, w2_ref, b2_ref, o_ref):
    x = x_ref[:, :64]
    h = jnp.dot(x, w1_ref[...], preferred_element_type=jnp.float32)
    h = jnp.maximum(h + b1_ref[...], 0.0)
    y = jnp.dot(h, w2_ref[...], preferred_element_type=jnp.float32)
    o_ref[...] = (y + b2_ref[...]).T


def kernel(user_id, table, W1, b1, W2, b2):
    batch = user_id.shape[0]
    emb_dim = table.shape[1]

    table_t = table.T
    idx = user_id.astype(jnp.int32)
    emb_pad = _make_sc_gather(batch, emb_dim)(table_t, idx)

    block = 2048
    grid = batch // block
    out_t = pl.pallas_call(
        _mlp_body,
        grid=(grid,),
        in_specs=[
            pl.BlockSpec((block, 2 * emb_dim), lambda i: (i, 0)),
            pl.BlockSpec((emb_dim, emb_dim), lambda i: (0, 0)),
            pl.BlockSpec((1, emb_dim), lambda i: (0, 0)),
            pl.BlockSpec((emb_dim, emb_dim), lambda i: (0, 0)),
            pl.BlockSpec((1, emb_dim), lambda i: (0, 0)),
        ],
        out_specs=pl.BlockSpec((emb_dim, block), lambda i: (0, i)),
        out_shape=jax.ShapeDtypeStruct((emb_dim, batch), jnp.float32),
    )(emb_pad, W1, b1.reshape(1, emb_dim), W2, b2.reshape(1, emb_dim))
    return out_t.T


# speedup vs baseline: 4.2714x; 1.7277x over previous
import functools

import jax
import jax.numpy as jnp
from jax import lax
from jax.experimental import pallas as pl
from jax.experimental.pallas import tpu as pltpu
from jax.experimental.pallas import tpu_sc as plsc

_NC = 2
_NS = 16
_NW = _NC * _NS
_L = 16

_ROWS = 1000001
_BATCH = 16384
_CHUNK = 512
_N_FULL = _ROWS // _CHUNK
_CH_PER_W = _N_FULL // _NW
_TAIL_LO = _N_FULL * _CHUNK
_TAIL_W = _ROWS - _TAIL_LO
_NBUF = 3

_HITS_CAP = 896
_HITS_DUMP = _HITS_CAP + _L
_WORK_CAP = 128
_WORK_DUMP = _WORK_CAP + _L
_STAGE = 32


def _lanes():
    return lax.broadcasted_iota(jnp.int32, (_L,), 0)


def _splat(x):
    return jnp.full((_L,), x, dtype=jnp.int32)


def _scalar_at(ref, w):
    base = (w // _L) * _L
    vec = ref[pl.ds(base, _L)]
    lane = w - base
    return jnp.max(jnp.where(_lanes() == lane, vec, jnp.int32(-1)))


def _collect_hits(idx_v, hits_r, hits_b, wid, count):
    def body(g, cnt):
        v = idx_v[pl.ds(g * _L, _L)]
        m = ((v >> 9) & (_NW - 1)) == wid
        mi = m.astype(jnp.int32)
        pos = jnp.where(m, cnt - 1 + jnp.cumsum(mi), _HITS_DUMP)
        plsc.store_scatter(hits_r, [pos], v)
        plsc.store_scatter(hits_b, [pos], _lanes() + g * _L)
        return jnp.minimum(cnt + jnp.sum(mi), _HITS_CAP)

    return lax.fori_loop(0, _BATCH // _L, body, count)


def _reset_scatteridx(scatteridx):
    for k in range(_STAGE // _L):
        scatteridx[pl.ds(k * _L, _L)] = _splat(_BATCH)


def _process_window(buf, win_lo, win_w, state, refs):
    (hits_r, hits_b, work_m, work_b, rowstage, scatteridx, out_hbm,
     scat_sem) = refs
    count, nrows = state

    def scan_body(h, wc):
        hv = hits_r[pl.ds(h * _L, _L)]
        bv = hits_b[pl.ds(h * _L, _L)]
        m = (hv >= win_lo) & (hv < win_lo + win_w)
        mi = m.astype(jnp.int32)
        pos = jnp.where(m, wc - 1 + jnp.cumsum(mi), _WORK_DUMP)
        plsc.store_scatter(work_m, [pos], hv - win_lo)
        plsc.store_scatter(work_b, [pos], bv)
        return jnp.minimum(wc + jnp.sum(mi), _WORK_CAP)

    n_win = (count + _L - 1) // _L
    wc = lax.fori_loop(0, n_win, scan_body, jnp.int32(0))

    def hit_body(w, nr):
        m_loc = _scalar_at(work_m, w)
        b = _scalar_at(work_b, w)
        for g in range(4):
            cvec = _lanes() + g * _L
            vals = plsc.load_gather(buf, [cvec, _splat(m_loc)])
            rowstage[nr, pl.ds(g * _L, _L)] = vals
        sbase = (nr // _L) * _L
        svec = scatteridx[pl.ds(sbase, _L)]
        scatteridx[pl.ds(sbase, _L)] = jnp.where(
            _lanes() == nr - sbase, b, svec
        )
        full = nr + 1 == _STAGE

        @pl.when(full)
        def _():
            pltpu.async_copy(rowstage, out_hbm.at[scatteridx], scat_sem).wait()
            _reset_scatteridx(scatteridx)

        return jnp.where(full, jnp.int32(0), nr + 1)

    nrows = lax.fori_loop(0, wc, hit_body, nrows)
    return count, nrows


def _make_sc_gather(batch, emb_dim):
    mesh = plsc.VectorSubcoreMesh(core_axis_name="c", subcore_axis_name="s")

    @functools.partial(
        pl.kernel,
        out_type=jax.ShapeDtypeStruct((batch + 8, 2 * emb_dim), jnp.float32),
        mesh=mesh,
        scratch_types=[
            pltpu.VMEM((batch,), jnp.int32),
            pltpu.VMEM((_NBUF, emb_dim, _CHUNK), jnp.float32),
            pltpu.VMEM((emb_dim, _TAIL_W), jnp.float32),
            pltpu.VMEM((_HITS_DUMP + _L,), jnp.int32),
            pltpu.VMEM((_HITS_DUMP + _L,), jnp.int32),
            pltpu.VMEM((_WORK_DUMP + _L,), jnp.int32),
            pltpu.VMEM((_WORK_DUMP + _L,), jnp.int32),
            pltpu.VMEM((_STAGE, 2 * emb_dim), jnp.float32),
            pltpu.VMEM((_STAGE,), jnp.int32),
            pltpu.SemaphoreType.DMA,
            pltpu.SemaphoreType.DMA,
        ],
        compiler_params=pltpu.CompilerParams(needs_layout_passes=False),
    )
    def gather(table_t_hbm, idx_hbm, out_hbm, idx_v, buf, tail_buf, hits_r,
               hits_b, work_m, work_b, rowstage, scatteridx, stream_sem,
               scat_sem):
        wid = lax.axis_index("s") * _NC + lax.axis_index("c")

        def chunk_lo(c):
            return (wid + _NW * c) * _CHUNK

        n_chunks = _CH_PER_W + (wid == 0).astype(jnp.int32)

        for p in range(_NBUF - 1):
            pltpu.async_copy(
                table_t_hbm.at[:, pl.ds((wid + _NW * p) * _CHUNK, _CHUNK)],
                buf.at[p], stream_sem,
            )

        pltpu.sync_copy(idx_hbm, idx_v)
        _reset_scatteridx(scatteridx)
        for k in range(0, _HITS_DUMP + _L, _L):
            hits_r[pl.ds(k, _L)] = _splat(jnp.int32(2**31 - 1))

        count = _collect_hits(idx_v, hits_r, hits_b, wid, jnp.int32(0))

        refs = (hits_r, hits_b, work_m, work_b, rowstage, scatteridx,
                out_hbm, scat_sem)

        def chunk_body(c, state):
            pltpu.make_async_copy(
                table_t_hbm.at[:, pl.ds(0, _CHUNK)], buf.at[c % _NBUF],
                stream_sem,
            ).wait()

            @pl.when(c + _NBUF - 1 < n_chunks)
            def _():
                pltpu.async_copy(
                    table_t_hbm.at[:, pl.ds(chunk_lo(c + _NBUF - 1), _CHUNK)],
                    buf.at[(c + _NBUF - 1) % _NBUF], stream_sem,
                )

            return _process_window(buf.at[c % _NBUF], chunk_lo(c), _CHUNK,
                                   state, refs)

        count, nrows = lax.fori_loop(0, n_chunks, chunk_body,
                                     (count, jnp.int32(0)))

        pltpu.sync_copy(table_t_hbm.at[:, pl.ds(_TAIL_LO, _TAIL_W)], tail_buf)
        _, nrows = _process_window(tail_buf, _TAIL_LO, _CHUNK, (count, nrows),
                                   refs)

        @pl.when(nrows > 0)
        def _():
            pltpu.async_copy(rowstage, out_hbm.at[scatteridx], scat_sem).wait()

    return gather


def _mlp_body(x_ref, w1_ref, b1_ref, w2_ref, b2_ref, o_ref):
    x = x_ref[:, :64]
    h = jnp.dot(x, w1_ref[...], preferred_element_type=jnp.float32)
    h = jnp.maximum(h + b1_ref[...], 0.0)
    y = jnp.dot(h, w2_ref[...], preferred_element_type=jnp.float32)
    o_ref[...] = (y + b2_ref[...]).T


def kernel(user_id, table, W1, b1, W2, b2):
    batch = user_id.shape[0]
    emb_dim = table.shape[1]

    table_t = table.T
    idx = user_id.astype(jnp.int32)
    emb_pad = _make_sc_gather(batch, emb_dim)(table_t, idx)

    block = 2048
    grid = batch // block
    out_t = pl.pallas_call(
        _mlp_body,
        grid=(grid,),
        in_specs=[
            pl.BlockSpec((block, 2 * emb_dim), lambda i: (i, 0)),
            pl.BlockSpec((emb_dim, emb_dim), lambda i: (0, 0)),
            pl.BlockSpec((1, emb_dim), lambda i: (0, 0)),
            pl.BlockSpec((emb_dim, emb_dim), lambda i: (0, 0)),
            pl.BlockSpec((1, emb_dim), lambda i: (0, 0)),
        ],
        out_specs=pl.BlockSpec((emb_dim, block), lambda i: (0, i)),
        out_shape=jax.ShapeDtypeStruct((emb_dim, batch), jnp.float32),
    )(emb_pad, W1, b1.reshape(1, emb_dim), W2, b2.reshape(1, emb_dim))
    return out_t.T
